# baseline (device time: 607416 ns/iter reference)
import jax
import jax.numpy as jnp
from jax import lax
from jax.experimental import pallas as pl
from jax.experimental.pallas import tpu as pltpu

P = 16
M = 4096
N = 8192
K = 256
CH = M // P
NR = 8
QN = N // NR
RIGHT = tuple(r < NR // 2 for r in range(NR))


def _rows(c):
    return pl.ds(c * CH, CH)


def _ar_body(x_ref, w_ref, out_ref, *s):
    k = NR
    acc = s[0:k]
    recv = s[k:2 * k]
    deq = s[2 * k:3 * k]
    qown = s[3 * k:4 * k]
    qrecv = s[4 * k:5 * k]
    amax_slots = s[5 * k]
    b = 5 * k + 1
    send_sems = s[b:b + k]
    recv_sems = s[b + k:b + 2 * k]
    qsend_sems = s[b + 2 * k:b + 3 * k]
    qrecv_sems = s[b + 3 * k:b + 4 * k]
    outs = s[b + 4 * k:b + 5 * k]
    amax_send_sems = s[b + 5 * k]
    amax_recv_sems = s[b + 5 * k + 1]
    credit = s[b + 5 * k + 2:b + 6 * k + 2]
    creditq = s[b + 6 * k + 2:b + 7 * k + 2]

    i = lax.axis_index("i")
    right = lax.rem(i + 1, P)
    left = lax.rem(i + P - 1, P)

    def nbr_out(r):
        return right if RIGHT[r] else left

    def nbr_in(r):
        return left if RIGHT[r] else right

    def rs_add_chunk(r, u):
        if RIGHT[r]:
            return lax.rem(i + 2 * P - u - 1, P)
        return lax.rem(i + u + 1, P)

    def ag_chunk(r, t):
        if RIGHT[r]:
            return lax.rem(i + P - t, P)
        return lax.rem(i + t, P)

    def own_chunk(r):
        return right if RIGHT[r] else left

    def col(r):
        return pl.ds(r * QN, QN)

    def gemm(c, r):
        a = x_ref[_rows(c), :]
        w_q = w_ref[:, r * QN:(r + 1) * QN]
        return jnp.dot(a, w_q,
                       preferred_element_type=jnp.float32).astype(jnp.bfloat16)

    def mk_rs(r, u):
        slot = u % 2
        return pltpu.make_async_remote_copy(
            src_ref=acc[r].at[slot], dst_ref=recv[r].at[slot],
            send_sem=send_sems[r].at[slot], recv_sem=recv_sems[r].at[slot],
            device_id=(nbr_out(r),), device_id_type=pl.DeviceIdType.MESH)

    def mk_ag(r, t):
        slot = t % 2
        src = qown[r] if t == 0 else qrecv[r].at[(t - 1) % 2]
        return pltpu.make_async_remote_copy(
            src_ref=src, dst_ref=qrecv[r].at[slot],
            send_sem=qsend_sems[r].at[slot], recv_sem=qrecv_sems[r].at[slot],
            device_id=(nbr_out(r),), device_id_type=pl.DeviceIdType.MESH)

    def mk_store(r, t):
        if t < 0:
            return pltpu.make_async_copy(
                acc[r].at[0], out_ref.at[_rows(own_chunk(r)), col(r)],
                outs[r].at[0])
        return pltpu.make_async_copy(
            deq[r].at[t % 2], out_ref.at[_rows(ag_chunk(r, t)), col(r)],
            outs[r].at[t % 2])

    def signal(sem, target):
        pl.semaphore_signal(sem, inc=1, device_id=(target,),
                            device_id_type=pl.DeviceIdType.MESH)

    barrier = pltpu.get_barrier_semaphore()
    for nbr in (left, right):
        signal(barrier, nbr)
    pl.semaphore_wait(barrier, 2)

    for r in range(NR):
        acc[r][0] = gemm(i, r)
    for r in range(NR):
        mk_rs(r, 0).start()

    for u in range(P - 1):
        for r in range(NR):
            g = gemm(rs_add_chunk(r, u), r)
            mk_rs(r, u).wait()
            acc[r][(u + 1) % 2] = recv[r][u % 2] + g
            if u <= P - 4:
                signal(credit[r], nbr_in(r))
            if u + 1 <= P - 2:
                if u + 1 >= 2:
                    pl.semaphore_wait(credit[r], 1)
                mk_rs(r, u + 1).start()

    my_amax = jnp.float32(0.0)
    for r in range(NR):
        my_amax = jnp.maximum(my_amax, jnp.max(acc[r][1].astype(jnp.float32)))
    amax_slots[pl.ds(i, 1)] = jnp.full((1, 8, 128), my_amax, jnp.float32)
    for d in range(P):
        @pl.when(d != i)
        def _():
            pltpu.make_async_remote_copy(
                src_ref=amax_slots.at[i], dst_ref=amax_slots.at[i],
                send_sem=amax_send_sems.at[d], recv_sem=amax_recv_sems.at[i],
                device_id=(d,), device_id_type=pl.DeviceIdType.MESH).start()
    for d in range(P):
        @pl.when(d != i)
        def _():
            w = pltpu.make_async_remote_copy(
                src_ref=amax_slots.at[i], dst_ref=amax_slots.at[d],
                send_sem=amax_send_sems.at[d], recv_sem=amax_recv_sems.at[d],
                device_id=(d,), device_id_type=pl.DeviceIdType.MESH)
            w.wait_send()
            w.wait_recv()

    g_amax = jnp.max(amax_slots[...])
    scale = g_amax / 127.0
    inv = 127.0 / g_amax

    for r in range(NR):
        q = jnp.clip(jnp.round(
            jnp.maximum(acc[r][1].astype(jnp.float32), 0.0) * inv),
            0.0, 127.0).astype(jnp.int8)
        qown[r][...] = q
        acc[r][0] = (q.astype(jnp.float32) * scale).astype(jnp.bfloat16)
    for r in range(NR):
        mk_ag(r, 0).start()
        mk_store(r, -1).start()

    for t in range(P - 1):
        for r in range(NR):
            mk_ag(r, t).wait()
            if 1 <= t <= P - 3:
                signal(creditq[r], nbr_in(r))
            d_val = (qrecv[r][t % 2].astype(jnp.float32)
                     * scale).astype(jnp.bfloat16)
            if t == 0:
                mk_store(r, -1).wait()
            elif t >= 2:
                mk_store(r, t - 2).wait()
            deq[r][t % 2] = d_val
            mk_store(r, t).start()
            if t + 1 <= P - 2:
                if t + 1 >= 2:
                    pl.semaphore_wait(creditq[r], 1)
                mk_ag(r, t + 1).start()

    for r in range(NR):
        mk_store(r, P - 3).wait()
        mk_store(r, P - 2).wait()


def _fused_gemm_ar(x_bf, w_bf):
    ring_vmem = lambda dt, lead: [pltpu.VMEM(lead + (CH, QN), dt)
                                  for _ in range(NR)]
    return pl.pallas_call(
        _ar_body,
        out_shape=jax.ShapeDtypeStruct((M, N), jnp.bfloat16),
        in_specs=[
            pl.BlockSpec(memory_space=pltpu.MemorySpace.VMEM),
            pl.BlockSpec(memory_space=pltpu.MemorySpace.VMEM),
        ],
        out_specs=pl.BlockSpec(memory_space=pl.ANY),
        scratch_shapes=(
            ring_vmem(jnp.bfloat16, (2,))
            + ring_vmem(jnp.bfloat16, (2,))
            + ring_vmem(jnp.bfloat16, (2,))
            + ring_vmem(jnp.int8, ())
            + ring_vmem(jnp.int8, (2,))
            + [pltpu.VMEM((P, 8, 128), jnp.float32)]
            + [pltpu.SemaphoreType.DMA((2,)) for _ in range(4 * NR)]
            + [pltpu.SemaphoreType.DMA((2,)) for _ in range(NR)]
            + [pltpu.SemaphoreType.DMA((P,)) for _ in range(2)]
            + [pltpu.SemaphoreType.REGULAR for _ in range(2 * NR)]
        ),
        compiler_params=pltpu.CompilerParams(
            collective_id=0, vmem_limit_bytes=64 * 1024 * 1024),
    )(x_bf, w_bf)


def kernel(x, w_mat):
    x_bf = x.astype(jnp.bfloat16)
    w_bf = w_mat.astype(jnp.bfloat16)
    return _fused_gemm_ar(x_bf, w_bf)


# device time: 600471 ns/iter; 1.0116x vs baseline; 1.0116x over previous
import jax
import jax.numpy as jnp
from jax import lax
from jax.experimental import pallas as pl
from jax.experimental.pallas import tpu as pltpu

P = 16
M = 4096
N = 8192
K = 256
CH = M // P
NR = 4
QN = N // NR
RIGHT = tuple(r < NR // 2 for r in range(NR))


def _rows(c):
    return pl.ds(c * CH, CH)


def _ar_body(x_ref, w_ref, out_ref, *s):
    xb, wb = s[-2], s[-1]
    k = NR
    acc = s[0:k]
    recv = s[k:2 * k]
    deq = s[2 * k:3 * k]
    qown = s[3 * k:4 * k]
    qrecv = s[4 * k:5 * k]
    amax_slots = s[5 * k]
    b = 5 * k + 1
    send_sems = s[b:b + k]
    recv_sems = s[b + k:b + 2 * k]
    qsend_sems = s[b + 2 * k:b + 3 * k]
    qrecv_sems = s[b + 3 * k:b + 4 * k]
    outs = s[b + 4 * k:b + 5 * k]
    amax_send_sems = s[b + 5 * k]
    amax_recv_sems = s[b + 5 * k + 1]
    credit = s[b + 5 * k + 2:b + 6 * k + 2]
    creditq = s[b + 6 * k + 2:b + 7 * k + 2]

    i = lax.axis_index("i")
    right = lax.rem(i + 1, P)
    left = lax.rem(i + P - 1, P)

    def nbr_out(r):
        return right if RIGHT[r] else left

    def nbr_in(r):
        return left if RIGHT[r] else right

    def rs_add_chunk(r, u):
        if RIGHT[r]:
            return lax.rem(i + 2 * P - u - 1, P)
        return lax.rem(i + u + 1, P)

    def ag_chunk(r, t):
        if RIGHT[r]:
            return lax.rem(i + P - t, P)
        return lax.rem(i + t, P)

    def own_chunk(r):
        return right if RIGHT[r] else left

    def col(r):
        return pl.ds(r * QN, QN)

    def gemm(c, r):
        a = xb[_rows(c), :]
        w_q = wb[:, r * QN:(r + 1) * QN]
        return jnp.dot(a, w_q,
                       preferred_element_type=jnp.float32).astype(jnp.bfloat16)

    def mk_rs(r, u):
        slot = u % 2
        return pltpu.make_async_remote_copy(
            src_ref=acc[r].at[slot], dst_ref=recv[r].at[slot],
            send_sem=send_sems[r].at[slot], recv_sem=recv_sems[r].at[slot],
            device_id=(nbr_out(r),), device_id_type=pl.DeviceIdType.MESH)

    def mk_ag(r, t):
        slot = t % 2
        src = qown[r] if t == 0 else qrecv[r].at[(t - 1) % 2]
        return pltpu.make_async_remote_copy(
            src_ref=src, dst_ref=qrecv[r].at[slot],
            send_sem=qsend_sems[r].at[slot], recv_sem=qrecv_sems[r].at[slot],
            device_id=(nbr_out(r),), device_id_type=pl.DeviceIdType.MESH)

    def mk_store(r, t):
        if t < 0:
            return pltpu.make_async_copy(
                acc[r].at[0], out_ref.at[_rows(own_chunk(r)), col(r)],
                outs[r].at[0])
        return pltpu.make_async_copy(
            deq[r].at[t % 2], out_ref.at[_rows(ag_chunk(r, t)), col(r)],
            outs[r].at[t % 2])

    def signal(sem, target):
        pl.semaphore_signal(sem, inc=1, device_id=(target,),
                            device_id_type=pl.DeviceIdType.MESH)

    barrier = pltpu.get_barrier_semaphore()
    for nbr in (left, right):
        signal(barrier, nbr)
    xb[...] = x_ref[...].astype(jnp.bfloat16)
    wb[...] = w_ref[...].astype(jnp.bfloat16)
    pl.semaphore_wait(barrier, 2)

    for r in range(NR):
        acc[r][0] = gemm(i, r)
    for r in range(NR):
        mk_rs(r, 0).start()

    for u in range(P - 1):
        for r in range(NR):
            g = gemm(rs_add_chunk(r, u), r)
            mk_rs(r, u).wait()
            acc[r][(u + 1) % 2] = recv[r][u % 2] + g
            if u <= P - 4:
                signal(credit[r], nbr_in(r))
            if u + 1 <= P - 2:
                if u + 1 >= 2:
                    pl.semaphore_wait(credit[r], 1)
                mk_rs(r, u + 1).start()

    my_amax = jnp.float32(0.0)
    for r in range(NR):
        my_amax = jnp.maximum(my_amax, jnp.max(acc[r][1].astype(jnp.float32)))
    amax_slots[pl.ds(i, 1)] = jnp.full((1, 8, 128), my_amax, jnp.float32)
    for d in range(P):
        @pl.when(d != i)
        def _():
            pltpu.make_async_remote_copy(
                src_ref=amax_slots.at[i], dst_ref=amax_slots.at[i],
                send_sem=amax_send_sems.at[d], recv_sem=amax_recv_sems.at[i],
                device_id=(d,), device_id_type=pl.DeviceIdType.MESH).start()
    for d in range(P):
        @pl.when(d != i)
        def _():
            w = pltpu.make_async_remote_copy(
                src_ref=amax_slots.at[i], dst_ref=amax_slots.at[d],
                send_sem=amax_send_sems.at[d], recv_sem=amax_recv_sems.at[d],
                device_id=(d,), device_id_type=pl.DeviceIdType.MESH)
            w.wait_send()
            w.wait_recv()

    g_amax = jnp.max(amax_slots[...])
    scale = g_amax / 127.0
    inv = 127.0 / g_amax

    for r in range(NR):
        q = jnp.clip(jnp.round(
            jnp.maximum(acc[r][1].astype(jnp.float32), 0.0) * inv),
            0.0, 127.0).astype(jnp.int8)
        qown[r][...] = q
        acc[r][0] = (q.astype(jnp.float32) * scale).astype(jnp.bfloat16)
    for r in range(NR):
        mk_ag(r, 0).start()
        mk_store(r, -1).start()

    for t in range(P - 1):
        for r in range(NR):
            mk_ag(r, t).wait()
            if 1 <= t <= P - 3:
                signal(creditq[r], nbr_in(r))
            d_val = (qrecv[r][t % 2].astype(jnp.float32)
                     * scale).astype(jnp.bfloat16)
            if t == 0:
                mk_store(r, -1).wait()
            elif t >= 2:
                mk_store(r, t - 2).wait()
            deq[r][t % 2] = d_val
            mk_store(r, t).start()
            if t + 1 <= P - 2:
                if t + 1 >= 2:
                    pl.semaphore_wait(creditq[r], 1)
                mk_ag(r, t + 1).start()

    for r in range(NR):
        mk_store(r, P - 3).wait()
        mk_store(r, P - 2).wait()


def _fused_gemm_ar(x_bf, w_bf):
    ring_vmem = lambda dt, lead: [pltpu.VMEM(lead + (CH, QN), dt)
                                  for _ in range(NR)]
    return pl.pallas_call(
        _ar_body,
        out_shape=jax.ShapeDtypeStruct((M, N), jnp.bfloat16),
        in_specs=[
            pl.BlockSpec(memory_space=pltpu.MemorySpace.VMEM),
            pl.BlockSpec(memory_space=pltpu.MemorySpace.VMEM),
        ],
        out_specs=pl.BlockSpec(memory_space=pl.ANY),
        scratch_shapes=(
            ring_vmem(jnp.bfloat16, (2,))
            + ring_vmem(jnp.bfloat16, (2,))
            + ring_vmem(jnp.bfloat16, (2,))
            + ring_vmem(jnp.int8, ())
            + ring_vmem(jnp.int8, (2,))
            + [pltpu.VMEM((P, 8, 128), jnp.float32)]
            + [pltpu.SemaphoreType.DMA((2,)) for _ in range(4 * NR)]
            + [pltpu.SemaphoreType.DMA((2,)) for _ in range(NR)]
            + [pltpu.SemaphoreType.DMA((P,)) for _ in range(2)]
            + [pltpu.SemaphoreType.REGULAR for _ in range(2 * NR)]
            + [pltpu.VMEM((M, K), jnp.bfloat16),
               pltpu.VMEM((K, N), jnp.bfloat16)]
        ),
        compiler_params=pltpu.CompilerParams(
            collective_id=0, vmem_limit_bytes=64 * 1024 * 1024),
    )(x_bf, w_bf)


def kernel(x, w_mat):
    return _fused_gemm_ar(x, w_mat)
